# Initial kernel scaffold; baseline (speedup 1.0000x reference)
import jax
import jax.numpy as jnp
from jax.experimental import pallas as pl


def kernel(h_i, v_i, d_iI, unit_r_iI, mapping, Wf1, bf1, Wf2, bf2, Wl1, bl1, Wl2, bl2, W0, b0, W1, b1, W2, b2):
    raise NotImplementedError("write your pallas kernel here")


# speedup vs baseline: 6.5163x; 6.5163x over previous
import functools

import jax
import jax.numpy as jnp
from jax import lax
from jax.experimental import pallas as pl
from jax.experimental.pallas import tpu as pltpu
from jax.experimental.pallas import tpu_sc as plsc

F = 128
R = 50
CUT = 5.0
N_NODES = 10000
N_PAD = 10240
EB = 1000
W = 40
NC, NS = 2, 16
STRIPE = N_PAD // NS

_HIGH = lax.Precision.HIGHEST


def _edge_body(offs_ref, wf1_ref, bf1_ref, wf2_ref, bf2_ref, w0_ref, b0_ref,
               w1_ref, b1_ref, k_ref, k3_ref, d_ref, h_ref, v_ref, r_ref,
               dh_ref, dv0_ref, dv1_ref, dv2_ref):
    d = d_ref[...]
    offs = offs_ref[...]
    delta = CUT / (R - 1)
    coeff = -0.5 / (delta * delta)
    g = jnp.exp(coeff * (d - offs) ** 2)
    x = jax.lax.dot_general(g, wf1_ref[...], (((1,), (0,)), ((), ())),
                            precision=_HIGH,
                            preferred_element_type=jnp.float32) + bf1_ref[...]
    x = jnp.maximum(x, 0.0) + jnp.log1p(jnp.exp(-jnp.abs(x))) - 0.6931471805599453
    filt = jax.lax.dot_general(x, wf2_ref[...], (((1,), (0,)), ((), ())),
                               precision=_HIGH,
                               preferred_element_type=jnp.float32) + bf2_ref[...]
    ei = h_ref[...] * filt
    e0 = jax.lax.dot_general(ei, w0_ref[...], (((1,), (0,)), ((), ())),
                             precision=_HIGH,
                             preferred_element_type=jnp.float32) + b0_ref[...]
    e1 = jax.lax.dot_general(ei, w1_ref[...], (((1,), (0,)), ((), ())),
                             precision=_HIGH,
                             preferred_element_type=jnp.float32) + b1_ref[...]
    dh_ref[...] = e1
    rr = jax.lax.dot_general(r_ref[...], k3_ref[...], (((1,), (0,)), ((), ())),
                             precision=_HIGH,
                             preferred_element_type=jnp.float32)
    e0r = jax.lax.dot_general(e0, k_ref[...], (((1,), (0,)), ((), ())),
                              preferred_element_type=jnp.float32)
    e1r = jax.lax.dot_general(e1, k_ref[...], (((1,), (0,)), ((), ())),
                              preferred_element_type=jnp.float32)
    dv = e0r * rr + e1r * v_ref[...]
    dv0_ref[...] = dv[:, 0:128]
    dv1_ref[...] = dv[:, 128:256]
    dv2_ref[...] = dv[:, 256:384]


def _sc_body(dh_hbm, dv0_hbm, dv1_hbm, dv2_hbm, map_hbm, z128_hbm, ones_hbm,
             pout_hbm, idx_v, rows_v, acc_sh):
    cid = lax.axis_index("c")
    sid = lax.axis_index("s")
    rpt = map_hbm.shape[1]
    wid = cid * NS + sid
    s0 = pl.multiple_of(sid * STRIPE, 8)
    pltpu.sync_copy(map_hbm.at[wid], idx_v)
    for chunk in range(5):
        pltpu.sync_copy(z128_hbm, acc_sh.at[pl.ds(s0, STRIPE)])
        plsc.subcore_barrier()
        if chunk < 4:
            src = (dh_hbm, dv0_hbm, dv1_hbm, dv2_hbm)[chunk]

            @pl.loop(0, rpt)
            def _(w):
                e0 = pl.multiple_of((wid * rpt + w) * W, 8)
                pltpu.sync_copy(src.at[pl.ds(e0, W)], rows_v)
                pltpu.sync_copy(rows_v, acc_sh.at[idx_v.at[w]], add=True)
        else:
            pltpu.sync_copy(ones_hbm, rows_v)

            @pl.loop(0, rpt)
            def _(w):
                pltpu.sync_copy(rows_v, acc_sh.at[idx_v.at[w]], add=True)

        plsc.subcore_barrier()
        pltpu.sync_copy(acc_sh.at[pl.ds(s0, STRIPE)],
                        pout_hbm.at[cid, chunk, pl.ds(s0, STRIPE)])


def _merge_body(p_ref, dh_ref, dv_ref):
    p = p_ref[...]
    s = p[0] + p[1]
    cnt = s[4, :, 0:1]
    inv = 1.0 / jnp.maximum(cnt, 1.0)
    dh_ref[...] = s[0] * inv
    dv_ref[...] = jnp.concatenate([s[1], s[2], s[3]], axis=1) * inv


def kernel(h_i, v_i, d_iI, unit_r_iI, mapping, Wf1, bf1, Wf2, bf2, Wl1, bl1,
           Wl2, bl2, W0, b0, W1, b1, W2, b2):
    E = h_i.shape[0]
    N = N_NODES
    f32 = jnp.float32

    v_flat = v_i.reshape(E, 3 * F)
    d2 = d_iI.reshape(E, 1)
    r_pad = jnp.pad(unit_r_iI, ((0, 0), (0, 5)))
    offs = jnp.linspace(0.0, CUT, R, dtype=f32).reshape(1, R)

    li = lax.broadcasted_iota(jnp.int32, (F, 3 * F), 1)
    fi = lax.broadcasted_iota(jnp.int32, (F, 3 * F), 0)
    K = (li // 3 == fi).astype(f32)
    li3 = lax.broadcasted_iota(jnp.int32, (8, 3 * F), 1)
    ci3 = lax.broadcasted_iota(jnp.int32, (8, 3 * F), 0)
    K3 = (li3 % 3 == ci3).astype(f32)

    nb = E // EB
    wspec = lambda shp: pl.BlockSpec(shp, lambda i: (0,) * len(shp))
    espec = lambda w: pl.BlockSpec((EB, w), lambda i: (i, 0))
    eout = jax.ShapeDtypeStruct((E, F), f32)
    dh_e, dv0_e, dv1_e, dv2_e = pl.pallas_call(
        _edge_body,
        grid=(nb,),
        in_specs=[wspec((1, R)), wspec((R, F)), wspec((1, F)), wspec((F, F)),
                  wspec((1, F)), wspec((F, F)), wspec((1, F)), wspec((F, F)),
                  wspec((1, F)), wspec((F, 3 * F)), wspec((8, 3 * F)),
                  espec(1), espec(F), espec(3 * F), espec(8)],
        out_specs=[espec(F), espec(F), espec(F), espec(F)],
        out_shape=[eout, eout, eout, eout],
    )(offs, Wf1, bf1.reshape(1, F), Wf2, bf2.reshape(1, F),
      W0, b0.reshape(1, F), W1, b1.reshape(1, F), K, K3,
      d2, h_i, v_flat, r_pad)

    rpt = E // (NC * NS * W)
    mapr = mapping.reshape(NC * NS, rpt, W)
    z128 = jnp.zeros((STRIPE, F), f32)
    ones128 = jnp.ones((W, F), f32)

    mesh = plsc.VectorSubcoreMesh(core_axis_name="c", subcore_axis_name="s")
    sc_call = pl.kernel(
        _sc_body, mesh=mesh,
        out_type=jax.ShapeDtypeStruct((NC, 5, N_PAD, F), f32),
        scratch_types=[pltpu.VMEM((rpt, W), jnp.int32),
                       pltpu.VMEM((W, F), f32),
                       pltpu.VMEM_SHARED((N_PAD, F), f32)],
    )
    pout = sc_call(dh_e, dv0_e, dv1_e, dv2_e, mapr, z128, ones128)

    BN = 1000
    dh_i, dv_i = pl.pallas_call(
        _merge_body,
        grid=(N // BN,),
        in_specs=[pl.BlockSpec((NC, 5, BN, F), lambda i: (0, 0, i, 0))],
        out_specs=[pl.BlockSpec((BN, F), lambda i: (i, 0)),
                   pl.BlockSpec((BN, 3 * F), lambda i: (i, 0))],
        out_shape=[jax.ShapeDtypeStruct((N, F), f32),
                   jax.ShapeDtypeStruct((N, 3 * F), f32)],
    )(pout)

    return dh_i, dv_i.reshape(N, F, 3)


# speedup vs baseline: 11.1903x; 1.7173x over previous
import functools

import jax
import jax.numpy as jnp
from jax import lax
from jax.experimental import pallas as pl
from jax.experimental.pallas import tpu as pltpu
from jax.experimental.pallas import tpu_sc as plsc

F = 128
R = 50
CUT = 5.0
N_NODES = 10000
N_PAD = 10240
EB = 1000
W2E = 80
WT = 40
NWIN = 62
NC, NS = 2, 16
STRIPE = N_PAD // NS

_HIGH = None


def _dot(a, b, prec=None):
    return jax.lax.dot_general(a, b, (((1,), (0,)), ((), ())),
                               precision=prec,
                               preferred_element_type=jnp.float32)


def _edge_body(offs_ref, wf1_ref, bf1_ref, wf2_ref, bf2_ref, w01_ref, b01_ref,
               kk_ref, k3_ref, d_ref, h_ref, v_ref, r_ref,
               dh_ref, dv0_ref, dv1_ref, dv2_ref):
    d = d_ref[...]
    offs = offs_ref[...]
    delta = CUT / (R - 1)
    coeff = -0.5 / (delta * delta)
    g = jnp.exp(coeff * (d - offs) ** 2)
    x = _dot(g, wf1_ref[...], _HIGH) + bf1_ref[...]
    x = jnp.maximum(x, 0.0) + jnp.log1p(jnp.exp(-jnp.abs(x))) - 0.6931471805599453
    filt = _dot(x, wf2_ref[...], _HIGH) + bf2_ref[...]
    ei = h_ref[...] * filt
    e01 = _dot(ei, w01_ref[...], _HIGH) + b01_ref[...]
    dh_ref[...] = e01[:, F:]
    rr = _dot(r_ref[...], k3_ref[...])
    er = _dot(e01, kk_ref[...])
    dv = er[:, :3 * F] * rr + er[:, 3 * F:] * v_ref[...]
    dv0_ref[...] = dv[:, 0:128]
    dv1_ref[...] = dv[:, 128:256]
    dv2_ref[...] = dv[:, 256:384]


def _sc_body(dh_hbm, dv0_hbm, dv1_hbm, dv2_hbm, mapm_hbm, mapt_hbm, z128_hbm,
             ones_hbm, pout_hbm, idx_v, idxt_v, rows_a, rows_b, acc_sh,
             sem_a, sem_b):
    cid = lax.axis_index("c")
    sid = lax.axis_index("s")
    wid = cid * NS + sid
    et = NWIN * W2E + WT
    ebase = wid * et
    s0 = pl.multiple_of(sid * STRIPE, 8)
    pltpu.sync_copy(mapm_hbm.at[wid], idx_v)
    pltpu.sync_copy(mapt_hbm.at[wid], idxt_v)
    for chunk in range(5):
        pltpu.sync_copy(z128_hbm, acc_sh.at[pl.ds(s0, STRIPE)])
        plsc.subcore_barrier()
        if chunk < 4:
            src = (dh_hbm, dv0_hbm, dv1_hbm, dv2_hbm)[chunk]

            def win(w):
                return src.at[pl.ds(pl.multiple_of(ebase + w * W2E, 8), W2E)]

            pltpu.make_async_copy(win(0), rows_a, sem_a).start()

            @pl.loop(0, NWIN // 2)
            def _(t):
                w = t * 2
                pltpu.make_async_copy(win(w + 1), rows_b, sem_b).start()
                pltpu.make_async_copy(win(w), rows_a, sem_a).wait()
                pltpu.sync_copy(rows_a, acc_sh.at[idx_v.at[w]], add=True)

                @pl.when(w + 2 < NWIN)
                def _():
                    pltpu.make_async_copy(win(w + 2), rows_a, sem_a).start()

                pltpu.make_async_copy(win(w + 1), rows_b, sem_b).wait()
                pltpu.sync_copy(rows_b, acc_sh.at[idx_v.at[w + 1]], add=True)

            offt = pl.multiple_of(ebase + NWIN * W2E, 8)
            pltpu.sync_copy(src.at[pl.ds(offt, WT)], rows_a.at[pl.ds(0, WT)])
            pltpu.sync_copy(rows_a.at[pl.ds(0, WT)],
                            acc_sh.at[idxt_v.at[0]], add=True)
        else:
            pltpu.sync_copy(ones_hbm, rows_a)

            @pl.loop(0, NWIN)
            def _(w):
                pltpu.sync_copy(rows_a, acc_sh.at[idx_v.at[w]], add=True)

            pltpu.sync_copy(rows_a.at[pl.ds(0, WT)],
                            acc_sh.at[idxt_v.at[0]], add=True)

        plsc.subcore_barrier()
        pltpu.sync_copy(acc_sh.at[pl.ds(s0, STRIPE)],
                        pout_hbm.at[cid, chunk, pl.ds(s0, STRIPE)])


def _merge_body(p_ref, dh_ref, dv_ref):
    p = p_ref[...]
    s = p[0] + p[1]
    cnt = s[4, :, 0:1]
    inv = 1.0 / jnp.maximum(cnt, 1.0)
    dh_ref[...] = s[0] * inv
    dv_ref[...] = jnp.concatenate([s[1], s[2], s[3]], axis=1) * inv


def kernel(h_i, v_i, d_iI, unit_r_iI, mapping, Wf1, bf1, Wf2, bf2, Wl1, bl1,
           Wl2, bl2, W0, b0, W1, b1, W2, b2):
    E = h_i.shape[0]
    N = N_NODES
    f32 = jnp.float32

    v_flat = v_i.reshape(E, 3 * F)
    d2 = d_iI.reshape(E, 1)
    r_pad = jnp.pad(unit_r_iI, ((0, 0), (0, 5)))
    offs = jnp.linspace(0.0, CUT, R, dtype=f32).reshape(1, R)

    li = lax.broadcasted_iota(jnp.int32, (F, 3 * F), 1)
    fi = lax.broadcasted_iota(jnp.int32, (F, 3 * F), 0)
    K = (li // 3 == fi).astype(f32)
    li3 = lax.broadcasted_iota(jnp.int32, (8, 3 * F), 1)
    ci3 = lax.broadcasted_iota(jnp.int32, (8, 3 * F), 0)
    K3 = (li3 % 3 == ci3).astype(f32)
    KK = jnp.zeros((2 * F, 6 * F), f32)
    KK = KK.at[:F, :3 * F].set(K).at[F:, 3 * F:].set(K)
    W01 = jnp.concatenate([W0, W1], axis=1)
    b01 = jnp.concatenate([b0, b1]).reshape(1, 2 * F)

    nb = E // EB
    wspec = lambda shp: pl.BlockSpec(shp, lambda i: (0,) * len(shp))
    espec = lambda w: pl.BlockSpec((EB, w), lambda i: (i, 0))
    eout = jax.ShapeDtypeStruct((E, F), f32)
    dh_e, dv0_e, dv1_e, dv2_e = pl.pallas_call(
        _edge_body,
        grid=(nb,),
        in_specs=[wspec((1, R)), wspec((R, F)), wspec((1, F)), wspec((F, F)),
                  wspec((1, F)), wspec((F, 2 * F)), wspec((1, 2 * F)),
                  wspec((2 * F, 6 * F)), wspec((8, 3 * F)),
                  espec(1), espec(F), espec(3 * F), espec(8)],
        out_specs=[espec(F), espec(F), espec(F), espec(F)],
        out_shape=[eout, eout, eout, eout],
    )(offs, Wf1, bf1.reshape(1, F), Wf2, bf2.reshape(1, F),
      W01, b01, KK, K3, d2, h_i, v_flat, r_pad)

    et = NWIN * W2E + WT
    mm = mapping.reshape(NC * NS, et)
    mapm = mm[:, :NWIN * W2E].reshape(NC * NS, NWIN, W2E)
    mapt = mm[:, NWIN * W2E:].reshape(NC * NS, 1, WT)
    z128 = jnp.zeros((STRIPE, F), f32)
    ones128 = jnp.ones((W2E, F), f32)

    mesh = plsc.VectorSubcoreMesh(core_axis_name="c", subcore_axis_name="s")
    sc_call = pl.kernel(
        _sc_body, mesh=mesh,
        out_type=jax.ShapeDtypeStruct((NC, 5, N_PAD, F), f32),
        scratch_types=[pltpu.VMEM((NWIN, W2E), jnp.int32),
                       pltpu.VMEM((1, WT), jnp.int32),
                       pltpu.VMEM((W2E, F), f32),
                       pltpu.VMEM((W2E, F), f32),
                       pltpu.VMEM_SHARED((N_PAD, F), f32),
                       pltpu.SemaphoreType.DMA,
                       pltpu.SemaphoreType.DMA],
    )
    pout = sc_call(dh_e, dv0_e, dv1_e, dv2_e, mapm, mapt, z128, ones128)

    BN = 1000
    dh_i, dv_i = pl.pallas_call(
        _merge_body,
        grid=(N // BN,),
        in_specs=[pl.BlockSpec((NC, 5, BN, F), lambda i: (0, 0, i, 0))],
        out_specs=[pl.BlockSpec((BN, F), lambda i: (i, 0)),
                   pl.BlockSpec((BN, 3 * F), lambda i: (i, 0))],
        out_shape=[jax.ShapeDtypeStruct((N, F), f32),
                   jax.ShapeDtypeStruct((N, 3 * F), f32)],
    )(pout)

    return dh_i, dv_i.reshape(N, F, 3)


# speedup vs baseline: 14.6798x; 1.3118x over previous
import functools

import jax
import jax.numpy as jnp
from jax import lax
from jax.experimental import pallas as pl
from jax.experimental.pallas import tpu as pltpu
from jax.experimental.pallas import tpu_sc as plsc

F = 128
R = 50
CUT = 5.0
N_NODES = 10000
N_PAD = 10240
EB = 1000
W2E = 80
WT = 40
NWIN = 62
NC, NS = 2, 16
STRIPE = N_PAD // NS

_HIGH = None


def _dot(a, b, prec=None):
    return jax.lax.dot_general(a, b, (((1,), (0,)), ((), ())),
                               precision=prec,
                               preferred_element_type=jnp.float32)


def _tdot(a, b):
    return jax.lax.dot_general(a, b, (((0,), (0,)), ((), ())),
                               precision=_HIGH,
                               preferred_element_type=jnp.float32)


def _edge_body(offs_ref, wf1_ref, bf1_ref, wf2_ref, bf2_ref, w01_ref, b01_ref,
               m_ref, d_ref, h_ref, vx_ref, vy_ref, vz_ref, r_ref,
               dh_ref, dvx_ref, dvy_ref, dvz_ref):
    d = d_ref[...].reshape(1, EB)
    offs = offs_ref[...]
    delta = CUT / (R - 1)
    coeff = -0.5 / (delta * delta)
    gT = jnp.exp(coeff * (d - offs) ** 2)
    x = _tdot(gT, wf1_ref[...]) + bf1_ref[...]
    x = jnp.maximum(x, 0.0) + jnp.log1p(jnp.exp(-jnp.abs(x))) - 0.6931471805599453
    filt = _dot(x, wf2_ref[...], _HIGH) + bf2_ref[...]
    ei = h_ref[...] * filt
    e01 = _dot(ei, w01_ref[...], _HIGH) + b01_ref[...]
    e0 = e01[:, :F]
    e1 = e01[:, F:]
    dh_ref[...] = e1
    rT = r_ref[...].reshape(3, EB)
    rb = _tdot(rT, m_ref[...])
    dvx_ref[...] = e0 * rb[:, :F] + e1 * vx_ref[...].reshape(EB, F)
    dvy_ref[...] = e0 * rb[:, F:2 * F] + e1 * vy_ref[...].reshape(EB, F)
    dvz_ref[...] = e0 * rb[:, 2 * F:] + e1 * vz_ref[...].reshape(EB, F)


def _sc_body(dh_hbm, dv0_hbm, dv1_hbm, dv2_hbm, mapm_hbm, mapt_hbm, z128_hbm,
             ones_hbm, pout_hbm, idx_v, idxt_v, rows_a, rows_b, acc_sh,
             sem_a, sem_b):
    cid = lax.axis_index("c")
    sid = lax.axis_index("s")
    wid = cid * NS + sid
    et = NWIN * W2E + WT
    ebase = wid * et
    s0 = pl.multiple_of(sid * STRIPE, 8)
    pltpu.sync_copy(mapm_hbm.at[wid], idx_v)
    pltpu.sync_copy(mapt_hbm.at[wid], idxt_v)
    for chunk in range(5):
        pltpu.sync_copy(z128_hbm, acc_sh.at[pl.ds(s0, STRIPE)])
        plsc.subcore_barrier()
        if chunk < 4:
            src = (dh_hbm, dv0_hbm, dv1_hbm, dv2_hbm)[chunk]

            def win(w):
                return src.at[pl.ds(pl.multiple_of(ebase + w * W2E, 8), W2E)]

            pltpu.make_async_copy(win(0), rows_a, sem_a).start()

            @pl.loop(0, NWIN // 2)
            def _(t):
                w = t * 2
                pltpu.make_async_copy(win(w + 1), rows_b, sem_b).start()
                pltpu.make_async_copy(win(w), rows_a, sem_a).wait()
                pltpu.sync_copy(rows_a, acc_sh.at[idx_v.at[w]], add=True)

                @pl.when(w + 2 < NWIN)
                def _():
                    pltpu.make_async_copy(win(w + 2), rows_a, sem_a).start()

                pltpu.make_async_copy(win(w + 1), rows_b, sem_b).wait()
                pltpu.sync_copy(rows_b, acc_sh.at[idx_v.at[w + 1]], add=True)

            offt = pl.multiple_of(ebase + NWIN * W2E, 8)
            pltpu.sync_copy(src.at[pl.ds(offt, WT)], rows_a.at[pl.ds(0, WT)])
            pltpu.sync_copy(rows_a.at[pl.ds(0, WT)],
                            acc_sh.at[idxt_v.at[0]], add=True)
        else:
            pltpu.sync_copy(ones_hbm, rows_a)

            @pl.loop(0, NWIN)
            def _(w):
                pltpu.sync_copy(rows_a, acc_sh.at[idx_v.at[w]], add=True)

            pltpu.sync_copy(rows_a.at[pl.ds(0, WT)],
                            acc_sh.at[idxt_v.at[0]], add=True)

        plsc.subcore_barrier()
        pltpu.sync_copy(acc_sh.at[pl.ds(s0, STRIPE)],
                        pout_hbm.at[cid, chunk, pl.ds(s0, STRIPE)])


def _merge_body(p_ref, bigk_ref, dh_ref, dv_ref):
    p = p_ref[...]
    s = p[0] + p[1]
    cnt = s[4, :, 0:1]
    inv = 1.0 / jnp.maximum(cnt, 1.0)
    dh_ref[...] = s[0] * inv
    sv = jnp.concatenate([s[1], s[2], s[3]], axis=1) * inv
    dv_ref[...] = _dot(sv, bigk_ref[...])


def kernel(h_i, v_i, d_iI, unit_r_iI, mapping, Wf1, bf1, Wf2, bf2, Wl1, bl1,
           Wl2, bl2, W0, b0, W1, b1, W2, b2):
    E = h_i.shape[0]
    N = N_NODES
    f32 = jnp.float32

    nb = E // EB
    v3 = v_i.transpose(0, 2, 1).reshape(E, 3, 1, F)
    d3 = d_iI.reshape(nb, 1, EB)
    r3 = unit_r_iI.T.reshape(3, nb, EB).transpose(1, 0, 2)
    offs = jnp.linspace(0.0, CUT, R, dtype=f32).reshape(R, 1)

    ci = lax.broadcasted_iota(jnp.int32, (3, 3 * F), 0)
    li = lax.broadcasted_iota(jnp.int32, (3, 3 * F), 1)
    M = (li // F == ci).astype(f32)
    cb = lax.broadcasted_iota(jnp.int32, (3 * F, 3 * F), 0)
    lb = lax.broadcasted_iota(jnp.int32, (3 * F, 3 * F), 1)
    BIGK = ((lb % 3 == cb // F) & (lb // 3 == cb % F)).astype(f32)
    W01 = jnp.concatenate([W0, W1], axis=1)
    b01 = jnp.concatenate([b0, b1]).reshape(1, 2 * F)

    wspec = lambda shp: pl.BlockSpec(shp, lambda i: (0,) * len(shp))
    espec = lambda w: pl.BlockSpec((EB, w), lambda i: (i, 0))
    vspec = lambda c: pl.BlockSpec((EB, 1, 1, F), lambda i, c=c: (i, c, 0, 0))
    eout = jax.ShapeDtypeStruct((E, F), f32)
    dh_e, dv0_e, dv1_e, dv2_e = pl.pallas_call(
        _edge_body,
        grid=(nb,),
        in_specs=[wspec((R, 1)), wspec((R, F)), wspec((1, F)), wspec((F, F)),
                  wspec((1, F)), wspec((F, 2 * F)), wspec((1, 2 * F)),
                  wspec((3, 3 * F)),
                  pl.BlockSpec((1, 1, EB), lambda i: (i, 0, 0)),
                  espec(F), vspec(0), vspec(1), vspec(2),
                  pl.BlockSpec((1, 3, EB), lambda i: (i, 0, 0))],
        out_specs=[espec(F), espec(F), espec(F), espec(F)],
        out_shape=[eout, eout, eout, eout],
    )(offs, Wf1, bf1.reshape(1, F), Wf2, bf2.reshape(1, F),
      W01, b01, M, d3, h_i, v3, v3, v3, r3)

    et = NWIN * W2E + WT
    mm = mapping.reshape(NC * NS, et)
    mapm = mm[:, :NWIN * W2E].reshape(NC * NS, NWIN, W2E)
    mapt = mm[:, NWIN * W2E:].reshape(NC * NS, 1, WT)
    z128 = jnp.zeros((STRIPE, F), f32)
    ones128 = jnp.ones((W2E, F), f32)

    mesh = plsc.VectorSubcoreMesh(core_axis_name="c", subcore_axis_name="s")
    sc_call = pl.kernel(
        _sc_body, mesh=mesh,
        out_type=jax.ShapeDtypeStruct((NC, 5, N_PAD, F), f32),
        scratch_types=[pltpu.VMEM((NWIN, W2E), jnp.int32),
                       pltpu.VMEM((1, WT), jnp.int32),
                       pltpu.VMEM((W2E, F), f32),
                       pltpu.VMEM((W2E, F), f32),
                       pltpu.VMEM_SHARED((N_PAD, F), f32),
                       pltpu.SemaphoreType.DMA,
                       pltpu.SemaphoreType.DMA],
    )
    pout = sc_call(dh_e, dv0_e, dv1_e, dv2_e, mapm, mapt, z128, ones128)

    BN = 1000
    dh_i, dv_i = pl.pallas_call(
        _merge_body,
        grid=(N // BN,),
        in_specs=[pl.BlockSpec((NC, 5, BN, F), lambda i: (0, 0, i, 0)),
                  pl.BlockSpec((3 * F, 3 * F), lambda i: (0, 0))],
        out_specs=[pl.BlockSpec((BN, F), lambda i: (i, 0)),
                   pl.BlockSpec((BN, 3 * F), lambda i: (i, 0))],
        out_shape=[jax.ShapeDtypeStruct((N, F), f32),
                   jax.ShapeDtypeStruct((N, 3 * F), f32)],
    )(pout, BIGK)

    return dh_i, dv_i.reshape(N, F, 3)
